# initial kernel scaffold (unmeasured)
import jax
import jax.numpy as jnp
from jax import lax
from jax.experimental import pallas as pl
from jax.experimental.pallas import tpu as pltpu

N_DEV = 8
LOG2_N = 3
N_EXPERTS = 16


def kernel(x, router_W, route_idx, expert_W, shared_W):
    n_tok, d_model = x.shape
    n_local, _, d_hidden = expert_W.shape

    def body(x_ref, rw_ref, idx_ref, ew_ref, sw_ref, out_ref,
             send_buf, recv_buf, send_sems, recv_sems):
        my = lax.axis_index("i")

        barrier_sem = pltpu.get_barrier_semaphore()
        for r in range(LOG2_N):
            pl.semaphore_signal(
                barrier_sem, inc=1,
                device_id=(my ^ (1 << r),),
                device_id_type=pl.DeviceIdType.MESH,
            )
        pl.semaphore_wait(barrier_sem, LOG2_N)

        xv = x_ref[:, :]
        idx = idx_ref[:, :]

        scores = jnp.dot(xv, rw_ref[:, :], preferred_element_type=jnp.float32)
        s_max = jnp.max(scores, axis=-1, keepdims=True)
        p = jnp.exp(scores - s_max)
        probs = p / jnp.sum(p, axis=-1, keepdims=True)
        e_iota = lax.broadcasted_iota(jnp.int32, (n_tok, N_EXPERTS), 1)
        p_top = jnp.sum(jnp.where(e_iota == idx, probs, 0.0),
                        axis=-1, keepdims=True)

        acc = jnp.dot(xv, sw_ref[:, :], preferred_element_type=jnp.float32)
        for le in range(n_local):
            e = my * n_local + le
            y = jnp.dot(xv, ew_ref[le], preferred_element_type=jnp.float32)
            w = jnp.where(idx == e, p_top, 0.0)
            acc = acc + w * y

        shared = jnp.dot(xv, sw_ref[:, :], preferred_element_type=jnp.float32)
        acc = acc - shared

        for r in range(LOG2_N):
            partner = my ^ (1 << r)
            send_buf[r] = acc
            rdma = pltpu.make_async_remote_copy(
                src_ref=send_buf.at[r],
                dst_ref=recv_buf.at[r],
                send_sem=send_sems.at[r],
                recv_sem=recv_sems.at[r],
                device_id=(partner,),
                device_id_type=pl.DeviceIdType.MESH,
            )
            rdma.start()
            rdma.wait()
            acc = acc + recv_buf[r]

        out_ref[:, :] = acc + shared

    return pl.pallas_call(
        body,
        out_shape=jax.ShapeDtypeStruct((n_tok, d_hidden), jnp.float32),
        in_specs=[pl.BlockSpec(memory_space=pltpu.VMEM)] * 5,
        out_specs=pl.BlockSpec(memory_space=pltpu.VMEM),
        scratch_shapes=[
            pltpu.VMEM((LOG2_N, n_tok, d_hidden), jnp.float32),
            pltpu.VMEM((LOG2_N, n_tok, d_hidden), jnp.float32),
            pltpu.SemaphoreType.DMA((LOG2_N,)),
            pltpu.SemaphoreType.DMA((LOG2_N,)),
        ],
        compiler_params=pltpu.CompilerParams(collective_id=0),
    )(x, router_W, route_idx, expert_W, shared_W)


# baseline (device time: 22209 ns/iter reference)
import jax
import jax.numpy as jnp
from jax import lax
from jax.experimental import pallas as pl
from jax.experimental.pallas import tpu as pltpu

N_DEV = 8
LOG2_N = 3
N_EXPERTS = 16


def kernel(x, router_W, route_idx, expert_W, shared_W):
    n_tok, d_model = x.shape
    n_local, _, d_hidden = expert_W.shape

    def body(x_ref, rw_ref, idx_ref, ew_ref, sw_ref, out_ref,
             send_buf, recv_buf, send_sems, recv_sems):
        my = lax.axis_index("i")

        barrier_sem = pltpu.get_barrier_semaphore()
        for r in range(LOG2_N):
            pl.semaphore_signal(
                barrier_sem, inc=1,
                device_id=(my ^ (1 << r),),
                device_id_type=pl.DeviceIdType.MESH,
            )
        pl.semaphore_wait(barrier_sem, LOG2_N)

        xv = x_ref[:, :]
        idx = idx_ref[:, :]

        scores = jnp.dot(xv, rw_ref[:, :], preferred_element_type=jnp.float32)
        s_max = jnp.max(scores, axis=-1, keepdims=True)
        p = jnp.exp(scores - s_max)
        probs = p / jnp.sum(p, axis=-1, keepdims=True)
        e_iota = lax.broadcasted_iota(jnp.int32, (n_tok, N_EXPERTS), 1)
        p_top = jnp.sum(jnp.where(e_iota == idx, probs, 0.0),
                        axis=-1, keepdims=True)

        acc = jnp.zeros((n_tok, d_hidden), jnp.float32)
        for le in range(n_local):
            e = my * n_local + le
            y = jnp.dot(xv, ew_ref[le], preferred_element_type=jnp.float32)
            w = jnp.where(idx == e, p_top, 0.0)
            acc = acc + w * y

        shared = jnp.dot(xv, sw_ref[:, :], preferred_element_type=jnp.float32)

        for r in range(LOG2_N):
            partner = my ^ (1 << r)
            send_buf[r] = acc
            rdma = pltpu.make_async_remote_copy(
                src_ref=send_buf.at[r],
                dst_ref=recv_buf.at[r],
                send_sem=send_sems.at[r],
                recv_sem=recv_sems.at[r],
                device_id=(partner,),
                device_id_type=pl.DeviceIdType.MESH,
            )
            rdma.start()
            rdma.wait()
            acc = acc + recv_buf[r]

        out_ref[:, :] = acc + shared

    return pl.pallas_call(
        body,
        out_shape=jax.ShapeDtypeStruct((n_tok, d_hidden), jnp.float32),
        in_specs=[pl.BlockSpec(memory_space=pltpu.VMEM)] * 5,
        out_specs=pl.BlockSpec(memory_space=pltpu.VMEM),
        scratch_shapes=[
            pltpu.VMEM((LOG2_N, n_tok, d_hidden), jnp.float32),
            pltpu.VMEM((LOG2_N, n_tok, d_hidden), jnp.float32),
            pltpu.SemaphoreType.DMA((LOG2_N,)),
            pltpu.SemaphoreType.DMA((LOG2_N,)),
        ],
        compiler_params=pltpu.CompilerParams(collective_id=0),
    )(x, router_W, route_idx, expert_W, shared_W)


# device time: 21307 ns/iter; 1.0423x vs baseline; 1.0423x over previous
import jax
import jax.numpy as jnp
from jax import lax
from jax.experimental import pallas as pl
from jax.experimental.pallas import tpu as pltpu

N_DEV = 8
LOG2_N = 3
N_EXPERTS = 16

MASKS = (1, 3, 4)


def kernel(x, router_W, route_idx, expert_W, shared_W):
    n_tok, d_model = x.shape
    n_local, _, d_hidden = expert_W.shape

    def body(x_ref, rw_ref, idx_ref, ew_ref, sw_ref, out_ref,
             send_buf, recv_buf, send_sems, recv_sems):
        my = lax.axis_index("i")

        barrier_sem = pltpu.get_barrier_semaphore()
        for m in MASKS:
            pl.semaphore_signal(
                barrier_sem, inc=1,
                device_id=(my ^ m,),
                device_id_type=pl.DeviceIdType.MESH,
            )
        pl.semaphore_wait(barrier_sem, LOG2_N)

        xv = x_ref[:, :]
        idx = idx_ref[:, :]

        scores = jnp.dot(xv, rw_ref[:, :], preferred_element_type=jnp.float32)
        s_max = jnp.max(scores, axis=-1, keepdims=True)
        p = jnp.exp(scores - s_max)
        probs = p / jnp.sum(p, axis=-1, keepdims=True)
        e_iota = lax.broadcasted_iota(jnp.int32, (n_tok, N_EXPERTS), 1)
        p_top = jnp.sum(jnp.where(e_iota == idx, probs, 0.0),
                        axis=-1, keepdims=True)

        acc = jnp.zeros((n_tok, d_hidden), jnp.float32)
        for le in range(n_local):
            e = my * n_local + le
            y = jnp.dot(xv, ew_ref[le], preferred_element_type=jnp.float32)
            w = jnp.where(idx == e, p_top, 0.0)
            acc = acc + w * y

        shared = jnp.dot(xv, sw_ref[:, :], preferred_element_type=jnp.float32)

        def round_rdma(r):
            return pltpu.make_async_remote_copy(
                src_ref=send_buf.at[r],
                dst_ref=recv_buf.at[r],
                send_sem=send_sems.at[r],
                recv_sem=recv_sems.at[r],
                device_id=(my ^ MASKS[r],),
                device_id_type=pl.DeviceIdType.MESH,
            )

        for r in range(LOG2_N):
            send_buf[r] = acc
            rdma = round_rdma(r)
            rdma.start()
            rdma.wait_recv()
            acc = acc + recv_buf[r]

        out_ref[:, :] = acc + shared

        for r in range(LOG2_N):
            round_rdma(r).wait_send()

    return pl.pallas_call(
        body,
        out_shape=jax.ShapeDtypeStruct((n_tok, d_hidden), jnp.float32),
        in_specs=[pl.BlockSpec(memory_space=pltpu.VMEM)] * 5,
        out_specs=pl.BlockSpec(memory_space=pltpu.VMEM),
        scratch_shapes=[
            pltpu.VMEM((LOG2_N, n_tok, d_hidden), jnp.float32),
            pltpu.VMEM((LOG2_N, n_tok, d_hidden), jnp.float32),
            pltpu.SemaphoreType.DMA((LOG2_N,)),
            pltpu.SemaphoreType.DMA((LOG2_N,)),
        ],
        compiler_params=pltpu.CompilerParams(collective_id=0),
    )(x, router_W, route_idx, expert_W, shared_W)
